# baseline (speedup 1.0000x reference)
import jax
import jax.numpy as jnp
from jax.experimental import pallas as pl
from jax.experimental.pallas import tpu as pltpu

NBUF = 12
LAG = 6


def _body(m_ref, ordv_ref, onpos_ref, non_ref, x_ref, o_ref,
          zbuf, bufs, rsem, wsem):
    bk = o_ref.shape[0]
    non = non_ref[0]

    def read(q, p):
        j = onpos_ref[q]
        pltpu.make_async_copy(x_ref.at[j], bufs.at[p], rsem.at[p]).start()

    for q in range(NBUF):
        @pl.when(q < non)
        def _(q=q):
            read(q, q)

    zbuf[...] = jnp.zeros_like(zbuf)

    for i in range(bk):
        on = m_ref[i] != 0

        @pl.when(on)
        def _(i=i):
            o = ordv_ref[i]
            p = jax.lax.rem(o, NBUF)
            pltpu.make_async_copy(x_ref.at[i], bufs.at[p], rsem.at[p]).wait()
            pltpu.make_async_copy(bufs.at[p], o_ref.at[i], wsem.at[i]).start()
            q2 = o + NBUF - LAG

            @pl.when(jnp.logical_and(o >= LAG, q2 < non))
            def _():
                jprev = onpos_ref[o - LAG]
                pltpu.make_async_copy(
                    bufs.at[jax.lax.rem(o - LAG, NBUF)],
                    o_ref.at[jprev], wsem.at[jprev]).wait()
                read(q2, jax.lax.rem(q2, NBUF))

        @pl.when(jnp.logical_not(on))
        def _(i=i):
            pltpu.make_async_copy(zbuf, o_ref.at[i], wsem.at[i]).start()

    for i in range(bk):
        pending = jnp.logical_or(m_ref[i] == 0, ordv_ref[i] >= non - NBUF)

        @pl.when(pending)
        def _(i=i):
            pltpu.make_async_copy(zbuf, o_ref.at[i], wsem.at[i]).wait()


def kernel(x, mask):
    B, K, C, H, W = x.shape
    BK = B * K
    x_r = x.reshape(BK, C, H, W)
    m = mask.reshape(BK).astype(jnp.int32)

    csum = jnp.cumsum(m)
    ordv = csum - m
    non = csum[-1:]
    idx = jnp.arange(BK, dtype=jnp.int32)
    key = jnp.where(m != 0, idx, BK + idx)
    onpos = jnp.argsort(key).astype(jnp.int32)

    y = pl.pallas_call(
        _body,
        in_specs=[
            pl.BlockSpec(memory_space=pltpu.SMEM),
            pl.BlockSpec(memory_space=pltpu.SMEM),
            pl.BlockSpec(memory_space=pltpu.SMEM),
            pl.BlockSpec(memory_space=pltpu.SMEM),
            pl.BlockSpec(memory_space=pl.ANY),
        ],
        out_specs=pl.BlockSpec(memory_space=pl.ANY),
        out_shape=jax.ShapeDtypeStruct((BK, C, H, W), x.dtype),
        scratch_shapes=[
            pltpu.VMEM((C, H, W), x.dtype),
            pltpu.VMEM((NBUF, C, H, W), x.dtype),
            pltpu.SemaphoreType.DMA((NBUF,)),
            pltpu.SemaphoreType.DMA((BK,)),
        ],
    )(m, ordv, onpos, non, x_r)
    return y.reshape(B, K * C, H, W)


# speedup vs baseline: 1.0448x; 1.0448x over previous
import jax
import jax.numpy as jnp
from jax.experimental import pallas as pl
from jax.experimental.pallas import tpu as pltpu

NBUF = 12
LAG = 6


def _body(m_ref, x_ref, o_ref, ordv, onpos, zbuf, bufs, rsem, wsem):
    bk = o_ref.shape[0]

    cnt = jnp.int32(0)
    for i in range(bk):
        mi = m_ref[i]
        ordv[i] = cnt

        @pl.when(mi != 0)
        def _(i=i, cnt=cnt):
            onpos[cnt] = i

        cnt = cnt + mi
    non = cnt

    def read(q, p):
        j = onpos[q]
        pltpu.make_async_copy(x_ref.at[j], bufs.at[p], rsem.at[p]).start()

    for q in range(NBUF):
        @pl.when(q < non)
        def _(q=q):
            read(q, q)

    zbuf[...] = jnp.zeros_like(zbuf)

    for i in range(bk):
        on = m_ref[i] != 0

        @pl.when(on)
        def _(i=i):
            o = ordv[i]
            p = jax.lax.rem(o, NBUF)
            pltpu.make_async_copy(x_ref.at[i], bufs.at[p], rsem.at[p]).wait()
            pltpu.make_async_copy(bufs.at[p], o_ref.at[i], wsem.at[i]).start()
            q2 = o + NBUF - LAG

            @pl.when(jnp.logical_and(o >= LAG, q2 < non))
            def _():
                jprev = onpos[o - LAG]
                pltpu.make_async_copy(
                    bufs.at[jax.lax.rem(o - LAG, NBUF)],
                    o_ref.at[jprev], wsem.at[jprev]).wait()
                read(q2, jax.lax.rem(q2, NBUF))

        @pl.when(jnp.logical_not(on))
        def _(i=i):
            pltpu.make_async_copy(zbuf, o_ref.at[i], wsem.at[i]).start()

    for i in range(bk):
        pending = jnp.logical_or(m_ref[i] == 0, ordv[i] >= non - NBUF)

        @pl.when(pending)
        def _(i=i):
            pltpu.make_async_copy(zbuf, o_ref.at[i], wsem.at[i]).wait()


def kernel(x, mask):
    B, K, C, H, W = x.shape
    BK = B * K
    x_r = x.reshape(BK, C, H, W)
    m = mask.reshape(BK).astype(jnp.int32)

    y = pl.pallas_call(
        _body,
        in_specs=[
            pl.BlockSpec(memory_space=pltpu.SMEM),
            pl.BlockSpec(memory_space=pl.ANY),
        ],
        out_specs=pl.BlockSpec(memory_space=pl.ANY),
        out_shape=jax.ShapeDtypeStruct((BK, C, H, W), x.dtype),
        scratch_shapes=[
            pltpu.SMEM((BK,), jnp.int32),
            pltpu.SMEM((BK,), jnp.int32),
            pltpu.VMEM((C, H, W), x.dtype),
            pltpu.VMEM((NBUF, C, H, W), x.dtype),
            pltpu.SemaphoreType.DMA((NBUF,)),
            pltpu.SemaphoreType.DMA((BK,)),
        ],
    )(m, x_r)
    return y.reshape(B, K * C, H, W)
